# baseline (speedup 1.0000x reference)
import functools

import jax
import jax.numpy as jnp
import numpy as np
from jax import lax
from jax.experimental import pallas as pl
from jax.experimental.pallas import tpu as pltpu
from jax.experimental.pallas import tpu_sc as plsc

NC = 2
NS = 16
NW = NC * NS
LANES = 16
BE = 48
TWS = 256
TW = 144


def _pre_tc(emb, W_pos_u, w_pos_beta, a_pos, W_neg_u, w_neg_beta, a_neg):
    n = emb.shape[0]
    ed = emb.shape[1]
    blk = 2000
    grid = (n // blk,)

    def body(ap_ref, an_ref, emb_ref, wup_ref, wbp_ref, wun_ref, wbn_ref,
             xsp_ref, up_ref, xsn_ref, un_ref):
        x = emb_ref[...]
        xsp_ref[...] = x * ap_ref[0]
        bp = jnp.tanh(jnp.dot(x, wbp_ref[...], preferred_element_type=jnp.float32))
        up_ref[...] = bp * jnp.dot(x, wup_ref[...],
                                   preferred_element_type=jnp.float32)
        xsn_ref[...] = x * an_ref[0]
        bn = jnp.tanh(jnp.dot(x, wbn_ref[...], preferred_element_type=jnp.float32))
        un_ref[...] = bn * jnp.dot(x, wun_ref[...],
                                   preferred_element_type=jnp.float32)

    outs = pl.pallas_call(
        body,
        grid=grid,
        in_specs=[
            pl.BlockSpec(memory_space=pltpu.SMEM),
            pl.BlockSpec(memory_space=pltpu.SMEM),
            pl.BlockSpec((blk, ed), lambda i: (i, 0)),
            pl.BlockSpec((ed, ed), lambda i: (0, 0)),
            pl.BlockSpec((ed, 1), lambda i: (0, 0)),
            pl.BlockSpec((ed, ed), lambda i: (0, 0)),
            pl.BlockSpec((ed, 1), lambda i: (0, 0)),
        ],
        out_specs=[
            pl.BlockSpec((blk, ed), lambda i: (i, 0)),
            pl.BlockSpec((blk, ed), lambda i: (i, 0)),
            pl.BlockSpec((blk, ed), lambda i: (i, 0)),
            pl.BlockSpec((blk, ed), lambda i: (i, 0)),
        ],
        out_shape=[
            jax.ShapeDtypeStruct((n, ed), jnp.float32),
            jax.ShapeDtypeStruct((n, ed), jnp.float32),
            jax.ShapeDtypeStruct((n, ed), jnp.float32),
            jax.ShapeDtypeStruct((n, ed), jnp.float32),
        ],
    )(a_pos.reshape(1), a_neg.reshape(1), emb,
      W_pos_u, w_pos_beta.reshape(ed, 1), W_neg_u, w_neg_beta.reshape(ed, 1))
    return outs


def _make_sc_phase(n_pad, e_pad):
    epw = e_pad // NS
    nb = epw // BE
    rps = n_pad // NS
    mesh = plsc.VectorSubcoreMesh(core_axis_name="c", subcore_axis_name="s")

    @functools.partial(
        pl.kernel,
        mesh=mesh,
        compiler_params=pltpu.CompilerParams(use_tc_tiling_on_sc=False,
                                             needs_layout_passes=False),
        out_type=jax.ShapeDtypeStruct((NC, n_pad, TW), jnp.float32),
        scratch_types=[
            pltpu.VMEM((2, BE), jnp.int32),
            pltpu.VMEM((2, BE), jnp.int32),
            pltpu.VMEM((2, BE), jnp.int32),
            pltpu.VMEM((2, BE, TWS), jnp.bfloat16),
            pltpu.VMEM((2, BE, 128), jnp.bfloat16),
            pltpu.VMEM((2, BE, TW), jnp.float32),
            pltpu.VMEM_SHARED((n_pad, TW), jnp.float32),
            pltpu.SemaphoreType.DMA,
            pltpu.SemaphoreType.DMA,
            pltpu.SemaphoreType.DMA,
            pltpu.SemaphoreType.DMA,
            pltpu.SemaphoreType.DMA,
            pltpu.SemaphoreType.DMA,
        ],
    )
    def sc_phase(st_hbm, dt_hbm, src_hbm, dst_hbm, zr_hbm, out_hbm,
                 si, di, ds2, sv, dv, ov, acc,
                 sg0, sg1, ss0, ss1, sx0, sx1):
        c = lax.axis_index("c")
        s = lax.axis_index("s")
        semg = (sg0, sg1)
        sems = (ss0, ss1)
        semi = (sx0, sx1)
        def ebase(b):
            return c * e_pad + s * epw + jnp.minimum(b, nb - 1) * BE

        def fire_idx(q, b):
            pltpu.async_copy(src_hbm.at[pl.ds(ebase(b), BE)], si.at[q],
                             semi[q])
            pltpu.async_copy(dst_hbm.at[pl.ds(ebase(b), BE)], di.at[q],
                             semi[q])

        def wait_idx(q, b):
            pltpu.make_async_copy(src_hbm.at[pl.ds(ebase(b), BE)], si.at[q],
                                  semi[q]).wait()
            pltpu.make_async_copy(dst_hbm.at[pl.ds(ebase(b), BE)], di.at[q],
                                  semi[q]).wait()

        def fire_gather(p, q):
            pltpu.async_copy(st_hbm.at[si.at[q]], sv.at[p], semg[p])
            pltpu.async_copy(dt_hbm.at[di.at[q]], dv.at[p], semg[p])

        def wait_gather(p, q):
            pltpu.make_async_copy(st_hbm.at[si.at[q]], sv.at[p], semg[p]).wait()
            pltpu.make_async_copy(dt_hbm.at[di.at[q]], dv.at[p], semg[p]).wait()

        def fire_scatter(p):
            pltpu.async_copy(ov.at[p], acc.at[ds2.at[p]], sems[p], add=True)

        def wait_scatter(p):
            pltpu.make_async_copy(ov.at[p], acc.at[ds2.at[p]], sems[p]).wait()

        def compute(p, q):
            for ch in range(BE // 16):
                ds2[p, pl.ds(ch * 16, 16)] = di[q, pl.ds(ch * 16, 16)]
            pp = jnp.full((LANES,), p, jnp.int32)
            c128 = jnp.full((LANES,), 128, jnp.int32)
            last = jnp.full((LANES,), LANES - 1, jnp.int32)
            lane_id = lax.iota(jnp.int32, LANES)
            z = jnp.zeros((LANES,), jnp.float32)
            fmt = plsc.PackFormat.INTERLEAVED
            co = c * 64
            for g in range(BE // LANES):
                gb = g * LANES
                lanes = lane_id + gb
                e2v = z
                asv = z
                for e in range(LANES):
                    r = gb + e
                    pe = None
                    pa = None
                    for t in range(2):
                        xsa, xsb = plsc.unpack(sv[p, r, pl.ds(32 * t, 32)],
                                               format=fmt)
                        xda, xdb = plsc.unpack(dv[p, r, pl.ds(co + 32 * t, 32)],
                                               format=fmt)
                        ua, ub = plsc.unpack(sv[p, r, pl.ds(64 + 32 * t, 32)],
                                             format=fmt)
                        dfa = xda - xsa
                        dfb = xdb - xsb
                        tpe = dfa * dfa + dfb * dfb
                        tpa = dfa * ua + dfb * ub
                        pe = tpe if t == 0 else pe + tpe
                        pa = tpa if t == 0 else pa + tpa
                    te = jnp.cumsum(pe).at[last].get(mode="promise_in_bounds")
                    ta = jnp.cumsum(pa).at[last].get(mode="promise_in_bounds")
                    sel = lane_id == e
                    e2v = jnp.where(sel, te, e2v)
                    asv = jnp.where(sel, ta, asv)
                x = e2v + 1e-12
                yi = plsc.bitcast(x, jnp.int32)
                y = plsc.bitcast((yi >> 1) + 0x1FBD1DF5, jnp.float32)
                y = 0.5 * (y + x / y)
                y = 0.5 * (y + x / y)
                y = 0.5 * (y + x / y)
                d = y + asv
                w = jnp.exp(-jnp.maximum(d, 0.0))
                plsc.store_scatter(ov, [pp, lanes, c128], w)
                for e in range(LANES):
                    r = gb + e
                    wb = w.at[jnp.full((LANES,), e, jnp.int32)].get(
                        mode="promise_in_bounds")
                    for t in range(4):
                        ha, hb = plsc.unpack(
                            sv[p, r, pl.ds(128 + 32 * t, 32)], format=fmt)
                        ov[p, r, pl.ds(32 * t, 16)] = ha * wb
                        ov[p, r, pl.ds(32 * t + 16, 16)] = hb * wb

        pltpu.sync_copy(zr_hbm, acc.at[pl.ds(s * rps, rps)])
        def zrow(r, _):
            ov[0, r, pl.ds(128, 16)] = jnp.zeros((16,), jnp.float32)
            ov[1, r, pl.ds(128, 16)] = jnp.zeros((16,), jnp.float32)
            return 0
        lax.fori_loop(0, BE, zrow, 0, unroll=4)
        plsc.subcore_barrier()

        pltpu.sync_copy(src_hbm.at[pl.ds(ebase(0), BE)], si.at[0])
        pltpu.sync_copy(dst_hbm.at[pl.ds(ebase(0), BE)], di.at[0])
        fire_gather(0, 0)
        fire_idx(1, 1)

        def pair(i, _):
            for j in range(2):
                b = i * 2 + j
                p = j
                wait_gather(p, p)
                wait_idx(1 - p, b + 1)
                fire_gather(1 - p, 1 - p)

                @pl.when(b >= 2)
                def _():
                    wait_scatter(p)

                compute(p, p)
                fire_scatter(p)
                fire_idx(p, b + 2)
            return 0

        lax.fori_loop(0, nb // 2, pair, 0)
        wait_scatter(0)
        wait_scatter(1)
        wait_gather(0, 0)
        wait_idx(1, nb + 1)
        plsc.subcore_barrier()
        pltpu.sync_copy(acc.at[pl.ds(s * rps, rps)],
                        out_hbm.at[c].at[pl.ds(s * rps, rps)])

    return sc_phase


def _post_tc(scp, scn, h_pad, wpt, wnt, wst, bp, bn, bs, n_pad):
    blk = n_pad // 8
    grid = (8,)

    def body(scp_ref, scn_ref, h_ref, wpt_ref, wnt_ref, wst_ref,
             bp_ref, bn_ref, bs_ref, o_ref):
        ap = scp_ref[:, :128]
        wsp = scp_ref[:, 128:129]
        an = scn_ref[:, :128]
        wsn = scn_ref[:, 128:129]
        msg = jnp.dot(ap, wpt_ref[...], preferred_element_type=jnp.float32)
        msg = msg + wsp * bp_ref[...]
        msg = msg + jnp.dot(an, wnt_ref[...], preferred_element_type=jnp.float32)
        msg = msg + wsn * bn_ref[...]
        msg = msg + jnp.dot(h_ref[...], wst_ref[...],
                            preferred_element_type=jnp.float32)
        msg = msg + bs_ref[...]
        o_ref[...] = jnp.maximum(msg, 0.0)

    return pl.pallas_call(
        body,
        grid=grid,
        in_specs=[
            pl.BlockSpec((blk, TW), lambda i: (i, 0)),
            pl.BlockSpec((blk, TW), lambda i: (i, 0)),
            pl.BlockSpec((blk, 128), lambda i: (i, 0)),
            pl.BlockSpec((128, 128), lambda i: (0, 0)),
            pl.BlockSpec((128, 128), lambda i: (0, 0)),
            pl.BlockSpec((128, 128), lambda i: (0, 0)),
            pl.BlockSpec((1, 128), lambda i: (0, 0)),
            pl.BlockSpec((1, 128), lambda i: (0, 0)),
            pl.BlockSpec((1, 128), lambda i: (0, 0)),
        ],
        out_specs=pl.BlockSpec((blk, 128), lambda i: (i, 0)),
        out_shape=jax.ShapeDtypeStruct((n_pad, 128), jnp.float32),
    )(scp, scn, h_pad, wpt, wnt, wst, bp, bn, bs)


def kernel(h, pos_edge_index, neg_edge_index, node_embeddings,
           pos_W_w, pos_W_b, neg_W_w, neg_W_b, self_W_w, self_W_b,
           w_pos_beta, W_pos_u, alpha_pos, w_neg_beta, W_neg_u, alpha_neg):
    n = h.shape[0]
    e = pos_edge_index.shape[1]
    n_pad = ((n + 16) + 127) // 128 * 128
    estep = NS * BE * 2
    e_pad = (e + estep - 1) // estep * estep
    rps = n_pad // NS

    a_pos = jnp.clip(alpha_pos, 0.1, 10.0)
    a_neg = jnp.clip(alpha_neg, 0.1, 10.0)

    xsp, up, xsn, un = _pre_tc(
        node_embeddings, W_pos_u, w_pos_beta, a_pos, W_neg_u, w_neg_beta, a_neg)

    rpad = n_pad - n
    h_pad = jnp.pad(h, ((0, rpad), (0, 0)))
    bf = jnp.bfloat16
    st_pos = jnp.concatenate(
        [jnp.pad(xsp, ((0, rpad), (0, 0))), jnp.pad(up, ((0, rpad), (0, 0))),
         h_pad], axis=1).astype(bf)
    dt_pos = jnp.pad(xsp, ((0, rpad), (0, 0))).astype(bf)
    st_neg = jnp.concatenate(
        [jnp.pad(xsn, ((0, rpad), (0, 0))), jnp.pad(un, ((0, rpad), (0, 0))),
         h_pad], axis=1).astype(bf)
    dt_neg = jnp.pad(xsn, ((0, rpad), (0, 0))).astype(bf)
    st_both = jnp.concatenate([st_pos, st_neg], axis=0)
    dt_both = jnp.concatenate([dt_pos, dt_neg], axis=1)
    zr = jnp.zeros((rps, TW), jnp.float32)

    def pad_edges(ei):
        epad = e_pad - e
        if epad == 0:
            return ei[0], ei[1]
        fill = jnp.full((epad,), n, jnp.int32)
        return (jnp.concatenate([ei[0], fill]), jnp.concatenate([ei[1], fill]))

    sp, dp = pad_edges(pos_edge_index)
    sn, dn = pad_edges(neg_edge_index)
    src_both = jnp.concatenate([sp, sn + n_pad])
    dst_both = jnp.concatenate([dp, dn])

    sc_phase = _make_sc_phase(n_pad, e_pad)
    scb = sc_phase(st_both, dt_both, src_both, dst_both, zr)
    scp = scb[0]
    scn = scb[1]

    perm = np.arange(128).reshape(4, 16, 2).transpose(0, 2, 1).reshape(-1)
    out = _post_tc(scp, scn, h_pad,
                   pos_W_w.T[perm], neg_W_w.T[perm], self_W_w.T,
                   pos_W_b.reshape(1, 128), neg_W_b.reshape(1, 128),
                   self_W_b.reshape(1, 128), n_pad)
    return out[:n]


# speedup vs baseline: 1.0200x; 1.0200x over previous
import functools

import jax
import jax.numpy as jnp
import numpy as np
from jax import lax
from jax.experimental import pallas as pl
from jax.experimental.pallas import tpu as pltpu
from jax.experimental.pallas import tpu_sc as plsc

NC = 2
NS = 16
NW = NC * NS
LANES = 16
BE = 48
TWS = 256
TW = 144


def _pre_tc(emb, W_pos_u, w_pos_beta, a_pos, W_neg_u, w_neg_beta, a_neg):
    n = emb.shape[0]
    ed = emb.shape[1]
    blk = 2000
    grid = (n // blk,)

    def body(ap_ref, an_ref, emb_ref, wup_ref, wbp_ref, wun_ref, wbn_ref,
             xsp_ref, up_ref, xsn_ref, un_ref):
        x = emb_ref[...]
        xsp_ref[...] = x * ap_ref[0]
        bp = jnp.tanh(jnp.dot(x, wbp_ref[...], preferred_element_type=jnp.float32))
        up_ref[...] = bp * jnp.dot(x, wup_ref[...],
                                   preferred_element_type=jnp.float32)
        xsn_ref[...] = x * an_ref[0]
        bn = jnp.tanh(jnp.dot(x, wbn_ref[...], preferred_element_type=jnp.float32))
        un_ref[...] = bn * jnp.dot(x, wun_ref[...],
                                   preferred_element_type=jnp.float32)

    outs = pl.pallas_call(
        body,
        grid=grid,
        in_specs=[
            pl.BlockSpec(memory_space=pltpu.SMEM),
            pl.BlockSpec(memory_space=pltpu.SMEM),
            pl.BlockSpec((blk, ed), lambda i: (i, 0)),
            pl.BlockSpec((ed, ed), lambda i: (0, 0)),
            pl.BlockSpec((ed, 1), lambda i: (0, 0)),
            pl.BlockSpec((ed, ed), lambda i: (0, 0)),
            pl.BlockSpec((ed, 1), lambda i: (0, 0)),
        ],
        out_specs=[
            pl.BlockSpec((blk, ed), lambda i: (i, 0)),
            pl.BlockSpec((blk, ed), lambda i: (i, 0)),
            pl.BlockSpec((blk, ed), lambda i: (i, 0)),
            pl.BlockSpec((blk, ed), lambda i: (i, 0)),
        ],
        out_shape=[
            jax.ShapeDtypeStruct((n, ed), jnp.float32),
            jax.ShapeDtypeStruct((n, ed), jnp.float32),
            jax.ShapeDtypeStruct((n, ed), jnp.float32),
            jax.ShapeDtypeStruct((n, ed), jnp.float32),
        ],
    )(a_pos.reshape(1), a_neg.reshape(1), emb,
      W_pos_u, w_pos_beta.reshape(ed, 1), W_neg_u, w_neg_beta.reshape(ed, 1))
    return outs


def _make_sc_phase(n_pad, e_pad):
    epw = e_pad // NS
    nb = epw // BE
    rps = n_pad // NS
    mesh = plsc.VectorSubcoreMesh(core_axis_name="c", subcore_axis_name="s")

    @functools.partial(
        pl.kernel,
        mesh=mesh,
        compiler_params=pltpu.CompilerParams(use_tc_tiling_on_sc=False,
                                             needs_layout_passes=False),
        out_type=jax.ShapeDtypeStruct((NC, n_pad, TW), jnp.float32),
        scratch_types=[
            pltpu.VMEM((2, BE), jnp.int32),
            pltpu.VMEM((2, BE), jnp.int32),
            pltpu.VMEM((2, BE), jnp.int32),
            pltpu.VMEM((2, BE, TWS), jnp.bfloat16),
            pltpu.VMEM((2, BE, 128), jnp.bfloat16),
            pltpu.VMEM((2, BE, TW), jnp.float32),
            pltpu.VMEM_SHARED((n_pad, TW), jnp.float32),
            pltpu.SemaphoreType.DMA,
            pltpu.SemaphoreType.DMA,
            pltpu.SemaphoreType.DMA,
            pltpu.SemaphoreType.DMA,
            pltpu.SemaphoreType.DMA,
            pltpu.SemaphoreType.DMA,
        ],
    )
    def sc_phase(st_hbm, dt_hbm, src_hbm, dst_hbm, zr_hbm, out_hbm,
                 si, di, ds2, sv, dv, ov, acc,
                 sg0, sg1, ss0, ss1, sx0, sx1):
        c = lax.axis_index("c")
        s = lax.axis_index("s")
        semg = (sg0, sg1)
        sems = (ss0, ss1)
        semi = (sx0, sx1)
        def ebase(b):
            return c * e_pad + s * epw + jnp.minimum(b, nb - 1) * BE

        def fire_idx(q, b):
            pltpu.async_copy(src_hbm.at[pl.ds(ebase(b), BE)], si.at[q],
                             semi[q])
            pltpu.async_copy(dst_hbm.at[pl.ds(ebase(b), BE)], di.at[q],
                             semi[q])

        def wait_idx(q, b):
            pltpu.make_async_copy(src_hbm.at[pl.ds(ebase(b), BE)], si.at[q],
                                  semi[q]).wait()
            pltpu.make_async_copy(dst_hbm.at[pl.ds(ebase(b), BE)], di.at[q],
                                  semi[q]).wait()

        HB = BE // 2

        def fire_gather(p, q):
            pltpu.async_copy(st_hbm.at[si.at[q, pl.ds(0, HB)]],
                             sv.at[p, pl.ds(0, HB)], semg[p])
            pltpu.async_copy(st_hbm.at[si.at[q, pl.ds(HB, HB)]],
                             sv.at[p, pl.ds(HB, HB)], semg[p])
            pltpu.async_copy(dt_hbm.at[di.at[q, pl.ds(0, HB)]],
                             dv.at[p, pl.ds(0, HB)], semg[p])
            pltpu.async_copy(dt_hbm.at[di.at[q, pl.ds(HB, HB)]],
                             dv.at[p, pl.ds(HB, HB)], semg[p])

        def wait_gather(p, q):
            pltpu.make_async_copy(st_hbm.at[si.at[q, pl.ds(0, HB)]],
                                  sv.at[p, pl.ds(0, HB)], semg[p]).wait()
            pltpu.make_async_copy(st_hbm.at[si.at[q, pl.ds(HB, HB)]],
                                  sv.at[p, pl.ds(HB, HB)], semg[p]).wait()
            pltpu.make_async_copy(dt_hbm.at[di.at[q, pl.ds(0, HB)]],
                                  dv.at[p, pl.ds(0, HB)], semg[p]).wait()
            pltpu.make_async_copy(dt_hbm.at[di.at[q, pl.ds(HB, HB)]],
                                  dv.at[p, pl.ds(HB, HB)], semg[p]).wait()

        def fire_scatter(p):
            pltpu.async_copy(ov.at[p], acc.at[ds2.at[p]], sems[p], add=True)

        def wait_scatter(p):
            pltpu.make_async_copy(ov.at[p], acc.at[ds2.at[p]], sems[p]).wait()

        def compute(p, q):
            for ch in range(BE // 16):
                ds2[p, pl.ds(ch * 16, 16)] = di[q, pl.ds(ch * 16, 16)]
            pp = jnp.full((LANES,), p, jnp.int32)
            c128 = jnp.full((LANES,), 128, jnp.int32)
            last = jnp.full((LANES,), LANES - 1, jnp.int32)
            lane_id = lax.iota(jnp.int32, LANES)
            z = jnp.zeros((LANES,), jnp.float32)
            fmt = plsc.PackFormat.INTERLEAVED
            co = c * 64
            for g in range(BE // LANES):
                gb = g * LANES
                lanes = lane_id + gb
                e2v = z
                asv = z
                for e in range(LANES):
                    r = gb + e
                    pe = None
                    pa = None
                    for t in range(2):
                        xsa, xsb = plsc.unpack(sv[p, r, pl.ds(32 * t, 32)],
                                               format=fmt)
                        xda, xdb = plsc.unpack(dv[p, r, pl.ds(co + 32 * t, 32)],
                                               format=fmt)
                        ua, ub = plsc.unpack(sv[p, r, pl.ds(64 + 32 * t, 32)],
                                             format=fmt)
                        dfa = xda - xsa
                        dfb = xdb - xsb
                        tpe = dfa * dfa + dfb * dfb
                        tpa = dfa * ua + dfb * ub
                        pe = tpe if t == 0 else pe + tpe
                        pa = tpa if t == 0 else pa + tpa
                    te = jnp.cumsum(pe).at[last].get(mode="promise_in_bounds")
                    ta = jnp.cumsum(pa).at[last].get(mode="promise_in_bounds")
                    sel = lane_id == e
                    e2v = jnp.where(sel, te, e2v)
                    asv = jnp.where(sel, ta, asv)
                x = e2v + 1e-12
                yi = plsc.bitcast(x, jnp.int32)
                y = plsc.bitcast((yi >> 1) + 0x1FBD1DF5, jnp.float32)
                y = 0.5 * (y + x / y)
                y = 0.5 * (y + x / y)
                y = 0.5 * (y + x / y)
                d = y + asv
                w = jnp.exp(-jnp.maximum(d, 0.0))
                plsc.store_scatter(ov, [pp, lanes, c128], w)
                for e in range(LANES):
                    r = gb + e
                    wb = w.at[jnp.full((LANES,), e, jnp.int32)].get(
                        mode="promise_in_bounds")
                    for t in range(4):
                        ha, hb = plsc.unpack(
                            sv[p, r, pl.ds(128 + 32 * t, 32)], format=fmt)
                        ov[p, r, pl.ds(32 * t, 16)] = ha * wb
                        ov[p, r, pl.ds(32 * t + 16, 16)] = hb * wb

        pltpu.sync_copy(zr_hbm, acc.at[pl.ds(s * rps, rps)])
        def zrow(r, _):
            ov[0, r, pl.ds(128, 16)] = jnp.zeros((16,), jnp.float32)
            ov[1, r, pl.ds(128, 16)] = jnp.zeros((16,), jnp.float32)
            return 0
        lax.fori_loop(0, BE, zrow, 0, unroll=4)
        plsc.subcore_barrier()

        pltpu.sync_copy(src_hbm.at[pl.ds(ebase(0), BE)], si.at[0])
        pltpu.sync_copy(dst_hbm.at[pl.ds(ebase(0), BE)], di.at[0])
        fire_gather(0, 0)
        fire_idx(1, 1)

        def pair(i, _):
            for j in range(2):
                b = i * 2 + j
                p = j
                wait_gather(p, p)
                wait_idx(1 - p, b + 1)
                fire_gather(1 - p, 1 - p)

                @pl.when(b >= 2)
                def _():
                    wait_scatter(p)

                compute(p, p)
                fire_scatter(p)
                fire_idx(p, b + 2)
            return 0

        lax.fori_loop(0, nb // 2, pair, 0)
        wait_scatter(0)
        wait_scatter(1)
        wait_gather(0, 0)
        wait_idx(1, nb + 1)
        plsc.subcore_barrier()
        pltpu.sync_copy(acc.at[pl.ds(s * rps, rps)],
                        out_hbm.at[c].at[pl.ds(s * rps, rps)])

    return sc_phase


def _post_tc(scp, scn, h_pad, wpt, wnt, wst, bp, bn, bs, n_pad):
    blk = n_pad // 8
    grid = (8,)

    def body(scp_ref, scn_ref, h_ref, wpt_ref, wnt_ref, wst_ref,
             bp_ref, bn_ref, bs_ref, o_ref):
        ap = scp_ref[:, :128]
        wsp = scp_ref[:, 128:129]
        an = scn_ref[:, :128]
        wsn = scn_ref[:, 128:129]
        msg = jnp.dot(ap, wpt_ref[...], preferred_element_type=jnp.float32)
        msg = msg + wsp * bp_ref[...]
        msg = msg + jnp.dot(an, wnt_ref[...], preferred_element_type=jnp.float32)
        msg = msg + wsn * bn_ref[...]
        msg = msg + jnp.dot(h_ref[...], wst_ref[...],
                            preferred_element_type=jnp.float32)
        msg = msg + bs_ref[...]
        o_ref[...] = jnp.maximum(msg, 0.0)

    return pl.pallas_call(
        body,
        grid=grid,
        in_specs=[
            pl.BlockSpec((blk, TW), lambda i: (i, 0)),
            pl.BlockSpec((blk, TW), lambda i: (i, 0)),
            pl.BlockSpec((blk, 128), lambda i: (i, 0)),
            pl.BlockSpec((128, 128), lambda i: (0, 0)),
            pl.BlockSpec((128, 128), lambda i: (0, 0)),
            pl.BlockSpec((128, 128), lambda i: (0, 0)),
            pl.BlockSpec((1, 128), lambda i: (0, 0)),
            pl.BlockSpec((1, 128), lambda i: (0, 0)),
            pl.BlockSpec((1, 128), lambda i: (0, 0)),
        ],
        out_specs=pl.BlockSpec((blk, 128), lambda i: (i, 0)),
        out_shape=jax.ShapeDtypeStruct((n_pad, 128), jnp.float32),
    )(scp, scn, h_pad, wpt, wnt, wst, bp, bn, bs)


def kernel(h, pos_edge_index, neg_edge_index, node_embeddings,
           pos_W_w, pos_W_b, neg_W_w, neg_W_b, self_W_w, self_W_b,
           w_pos_beta, W_pos_u, alpha_pos, w_neg_beta, W_neg_u, alpha_neg):
    n = h.shape[0]
    e = pos_edge_index.shape[1]
    n_pad = ((n + 16) + 127) // 128 * 128
    estep = NS * BE * 2
    e_pad = (e + estep - 1) // estep * estep
    rps = n_pad // NS

    a_pos = jnp.clip(alpha_pos, 0.1, 10.0)
    a_neg = jnp.clip(alpha_neg, 0.1, 10.0)

    xsp, up, xsn, un = _pre_tc(
        node_embeddings, W_pos_u, w_pos_beta, a_pos, W_neg_u, w_neg_beta, a_neg)

    rpad = n_pad - n
    h_pad = jnp.pad(h, ((0, rpad), (0, 0)))
    bf = jnp.bfloat16
    st_pos = jnp.concatenate(
        [jnp.pad(xsp, ((0, rpad), (0, 0))), jnp.pad(up, ((0, rpad), (0, 0))),
         h_pad], axis=1).astype(bf)
    dt_pos = jnp.pad(xsp, ((0, rpad), (0, 0))).astype(bf)
    st_neg = jnp.concatenate(
        [jnp.pad(xsn, ((0, rpad), (0, 0))), jnp.pad(un, ((0, rpad), (0, 0))),
         h_pad], axis=1).astype(bf)
    dt_neg = jnp.pad(xsn, ((0, rpad), (0, 0))).astype(bf)
    st_both = jnp.concatenate([st_pos, st_neg], axis=0)
    dt_both = jnp.concatenate([dt_pos, dt_neg], axis=1)
    zr = jnp.zeros((rps, TW), jnp.float32)

    def pad_edges(ei):
        epad = e_pad - e
        if epad == 0:
            return ei[0], ei[1]
        fill = jnp.full((epad,), n, jnp.int32)
        return (jnp.concatenate([ei[0], fill]), jnp.concatenate([ei[1], fill]))

    sp, dp = pad_edges(pos_edge_index)
    sn, dn = pad_edges(neg_edge_index)
    src_both = jnp.concatenate([sp, sn + n_pad])
    dst_both = jnp.concatenate([dp, dn])

    sc_phase = _make_sc_phase(n_pad, e_pad)
    scb = sc_phase(st_both, dt_both, src_both, dst_both, zr)
    scp = scb[0]
    scn = scb[1]

    perm = np.arange(128).reshape(4, 16, 2).transpose(0, 2, 1).reshape(-1)
    out = _post_tc(scp, scn, h_pad,
                   pos_W_w.T[perm], neg_W_w.T[perm], self_W_w.T,
                   pos_W_b.reshape(1, 128), neg_W_b.reshape(1, 128),
                   self_W_b.reshape(1, 128), n_pad)
    return out[:n]
